# baseline (speedup 1.0000x reference)
import functools

import jax
import jax.numpy as jnp
from jax import lax
from jax.experimental import pallas as pl
from jax.experimental.pallas import tpu as pltpu
from jax.experimental.pallas import tpu_sc as plsc

_NC = 2
_NS = 16
_NW = _NC * _NS
_CHUNK = 128


def _sc_gather(tok2d, emb, n_chunks):
    m = tok2d.shape[0] * tok2d.shape[1]
    d = emb.shape[1]
    b_per_w = n_chunks * _CHUNK
    mesh = plsc.VectorSubcoreMesh(core_axis_name="c", subcore_axis_name="s")

    @functools.partial(
        pl.kernel,
        mesh=mesh,
        out_type=jax.ShapeDtypeStruct((m, d), jnp.float32),
        scratch_types=[
            pltpu.VMEM((n_chunks, _CHUNK), jnp.int32),
            pltpu.VMEM((b_per_w, d), jnp.float32),
            pltpu.SemaphoreType.DMA,
        ],
        compiler_params=pltpu.CompilerParams(use_tc_tiling_on_sc=False),
    )
    def gather_kernel(tok_hbm, table_hbm, out_hbm, idx_v, rows_v, sem):
        wid = lax.axis_index("s") * _NC + lax.axis_index("c")
        pltpu.sync_copy(tok_hbm.at[pl.ds(wid * n_chunks, n_chunks)], idx_v)
        copies = []
        for j in range(n_chunks):
            copies.append(
                pltpu.async_copy(
                    table_hbm.at[idx_v.at[j]],
                    rows_v.at[pl.ds(j * _CHUNK, _CHUNK)],
                    sem,
                )
            )
        for c in copies:
            c.wait()
        pltpu.sync_copy(rows_v, out_hbm.at[pl.ds(wid * b_per_w, b_per_w)])

    return gather_kernel(tok2d, emb)


def _tc_matmul(ve, w, block_m):
    m, d = ve.shape
    dm = w.shape[0]

    def mm_body(ve_ref, w_ref, out_ref):
        out_ref[...] = lax.dot_general(
            ve_ref[...],
            w_ref[...],
            (((1,), (1,)), ((), ())),
            preferred_element_type=jnp.float32,
        )

    return pl.pallas_call(
        mm_body,
        grid=(m // block_m,),
        in_specs=[
            pl.BlockSpec((block_m, d), lambda i: (i, 0)),
            pl.BlockSpec((dm, d), lambda i: (0, 0)),
        ],
        out_specs=pl.BlockSpec((block_m, dm), lambda i: (i, 0)),
        out_shape=jax.ShapeDtypeStruct((m, dm), jnp.float32),
    )(ve, w)


def kernel(tokens, emb, W):
    b, s = tokens.shape
    m = b * s
    dm = W.shape[0]
    n_chunks = m // (_NW * _CHUNK)
    tok2d = tokens.reshape(_NW * n_chunks, _CHUNK).astype(jnp.int32)
    ve = _sc_gather(tok2d, emb, n_chunks)
    out = _tc_matmul(ve, W, 2048)
    return out.reshape(b, s, dm)
